# Initial kernel scaffold; baseline (speedup 1.0000x reference)
import jax
import jax.numpy as jnp
from jax.experimental import pallas as pl


def kernel(router_logits, correction_bias):
    raise NotImplementedError("write your pallas kernel here")


# speedup vs baseline: 24.9966x; 24.9966x over previous
import functools

import jax
import jax.numpy as jnp
from jax import lax
from jax.experimental import pallas as pl

N_EXPERTS = 256
N_GROUP = 8
GROUP_SIZE = N_EXPERTS // N_GROUP
TOPK_GROUP = 4
TOP_K = 8
ROUTED_SCALING = 2.5
NUM_TOKENS = 8192

BT = 256


def _router_block(x_ref, b_ref, idx_ref, w_ref):
    x = x_ref[...]
    s = 1.0 / (1.0 + jnp.exp(-x))
    bias = jnp.broadcast_to(b_ref[0:1, :], (BT, N_EXPERTS))
    sc = s + bias

    e_iota = lax.broadcasted_iota(jnp.int32, (BT, N_EXPERTS), 1)
    g_of_e = lax.shift_right_logical(e_iota, 5)

    neg_inf = jnp.float32(-jnp.inf)
    gs_cols = []
    for g in range(N_GROUP):
        xg = jnp.where(g_of_e == g, sc, neg_inf)
        m1 = jnp.max(xg, axis=-1, keepdims=True)
        im = jnp.where(xg == m1, e_iota, N_EXPERTS)
        l1 = jnp.min(im, axis=-1, keepdims=True)
        xg2 = jnp.where(e_iota == l1, neg_inf, xg)
        m2 = jnp.max(xg2, axis=-1, keepdims=True)
        gs_cols.append(m1 + m2)
    gs = jnp.concatenate(gs_cols, axis=1)

    a = gs[:, None, :]
    b = gs[:, :, None]
    h_iota = lax.broadcasted_iota(jnp.int32, (BT, N_GROUP, N_GROUP), 2)
    g_iota = lax.broadcasted_iota(jnp.int32, (BT, N_GROUP, N_GROUP), 1)
    beats = (a > b) | ((a == b) & (h_iota < g_iota))
    rank = jnp.sum(jnp.where(beats, 1.0, 0.0), axis=2)

    rk = jnp.zeros((BT, N_EXPERTS), dtype=jnp.float32)
    for g in range(N_GROUP):
        rg = jnp.broadcast_to(rank[:, g:g + 1], (BT, N_EXPERTS))
        rk = jnp.where(g_of_e == g, rg, rk)
    work = jnp.where(rk < float(TOPK_GROUP), sc, 0.0)

    idx_cols, sv_cols = [], []
    for _ in range(TOP_K):
        m = jnp.max(work, axis=-1, keepdims=True)
        im = jnp.where(work == m, e_iota, N_EXPERTS)
        l = jnp.min(im, axis=-1, keepdims=True)
        onehot = e_iota == l
        sv = jnp.sum(jnp.where(onehot, s, 0.0), axis=-1, keepdims=True)
        idx_cols.append(l)
        sv_cols.append(sv)
        work = jnp.where(onehot, neg_inf, work)
    idx = jnp.concatenate(idx_cols, axis=1)
    w = jnp.concatenate(sv_cols, axis=1)
    w = w / (jnp.sum(w, axis=-1, keepdims=True) + 1e-20) * ROUTED_SCALING

    idx_ref[...] = idx
    w_ref[...] = w


@jax.jit
def _router(router_logits, bias2d):
    grid = (NUM_TOKENS // BT,)
    return pl.pallas_call(
        _router_block,
        grid=grid,
        in_specs=[
            pl.BlockSpec((BT, N_EXPERTS), lambda i: (i, 0)),
            pl.BlockSpec((8, N_EXPERTS), lambda i: (0, 0)),
        ],
        out_specs=[
            pl.BlockSpec((BT, TOP_K), lambda i: (i, 0)),
            pl.BlockSpec((BT, TOP_K), lambda i: (i, 0)),
        ],
        out_shape=[
            jax.ShapeDtypeStruct((NUM_TOKENS, TOP_K), jnp.int32),
            jax.ShapeDtypeStruct((NUM_TOKENS, TOP_K), jnp.float32),
        ],
    )(router_logits, bias2d)


def kernel(router_logits, correction_bias):
    bias2d = jnp.broadcast_to(correction_bias[None, :], (8, N_EXPERTS))
    idx, w = _router(router_logits, bias2d)
    return idx, w
